# baseline (speedup 1.0000x reference)
import functools

import jax
import jax.numpy as jnp
from jax import lax
from jax.experimental import pallas as pl
from jax.experimental.pallas import tpu as pltpu
from jax.experimental.pallas import tpu_sc as plsc

_B = 16384
_EMBED = 32
_NC = 2
_NS = 16
_NW = _NC * _NS
_BPW = _B // _NW
_CHUNK = 128
_NCHUNK = _BPW // _CHUNK


def _sc_gather_body(utab, mtab, uidx, midx, uemb, memb,
                    idx_v, rows_a, rows_b, sem_a, sem_b):
  wid = lax.axis_index("s") * _NC + lax.axis_index("c")
  base = wid * _BPW

  def gather_table(tab, idx_hbm, out_hbm):
    pltpu.sync_copy(idx_hbm.at[wid], idx_v)
    cp_a = pltpu.async_copy(tab.at[idx_v.at[0]], rows_a, sem_a)
    for j in range(_NCHUNK):
      nxt = None
      if j + 1 < _NCHUNK:
        nxt = pltpu.async_copy(
            tab.at[idx_v.at[j + 1]], rows_b if j % 2 == 0 else rows_a,
            sem_b if j % 2 == 0 else sem_a)
      cp_a.wait()
      rows = rows_a if j % 2 == 0 else rows_b
      pltpu.sync_copy(rows, out_hbm.at[pl.ds(base + j * _CHUNK, _CHUNK)])
      cp_a = nxt

  gather_table(utab, uidx, uemb)
  gather_table(mtab, midx, memb)


@functools.partial(jax.jit, static_argnames=())
def _sc_gather(user_table, movie_table, uidx, midx):
  mesh = plsc.VectorSubcoreMesh(core_axis_name="c", subcore_axis_name="s")
  f = pl.kernel(
      _sc_gather_body,
      out_type=(
          jax.ShapeDtypeStruct((_B, _EMBED), jnp.float32),
          jax.ShapeDtypeStruct((_B, _EMBED), jnp.float32),
      ),
      mesh=mesh,
      compiler_params=pltpu.CompilerParams(use_tc_tiling_on_sc=False),
      scratch_types=[
          pltpu.VMEM((_NCHUNK, _CHUNK), jnp.int32),
          pltpu.VMEM((_CHUNK, _EMBED), jnp.float32),
          pltpu.VMEM((_CHUNK, _EMBED), jnp.float32),
          pltpu.SemaphoreType.DMA,
          pltpu.SemaphoreType.DMA,
      ],
  )
  return f(user_table, movie_table, uidx, midx)


def _mlp_body(u_ref, m_ref, w1u_ref, w1m_ref, b1_ref, w2_ref, b2_ref,
              w3_ref, b3_ref, o_ref):
  h = (jnp.dot(u_ref[...], w1u_ref[...], preferred_element_type=jnp.float32)
       + jnp.dot(m_ref[...], w1m_ref[...], preferred_element_type=jnp.float32)
       + b1_ref[...])
  h = jnp.maximum(h, 0.0)
  h = jnp.dot(h, w2_ref[...], preferred_element_type=jnp.float32) + b2_ref[...]
  h = jnp.maximum(h, 0.0)
  y = jnp.dot(h, w3_ref[...], preferred_element_type=jnp.float32) + b3_ref[...]
  o_ref[...] = jax.nn.sigmoid(y)


_MLP_BLOCK = 2048


def _mlp(uemb, memb, w1u, w1m, b1, w2, b2, w3, b3):
  grid = (_B // _MLP_BLOCK,)
  full = lambda shape: pl.BlockSpec(shape, lambda i: (0, 0))
  return pl.pallas_call(
      _mlp_body,
      grid=grid,
      in_specs=[
          pl.BlockSpec((_MLP_BLOCK, _EMBED), lambda i: (i, 0)),
          pl.BlockSpec((_MLP_BLOCK, _EMBED), lambda i: (i, 0)),
          full(w1u.shape), full(w1m.shape), full(b1.shape),
          full(w2.shape), full(b2.shape), full(w3.shape), full(b3.shape),
      ],
      out_specs=pl.BlockSpec((_MLP_BLOCK, 1), lambda i: (i, 0)),
      out_shape=jax.ShapeDtypeStruct((_B, 1), jnp.float32),
  )(uemb, memb, w1u, w1m, b1, w2, b2, w3, b3)


def kernel(user_idx, movie_idx, user_table, movie_table, W1, b1, W2, b2, W3, b3):
  uidx = user_idx.astype(jnp.int32).reshape(_NW, _NCHUNK, _CHUNK)
  midx = movie_idx.astype(jnp.int32).reshape(_NW, _NCHUNK, _CHUNK)
  uemb, memb = _sc_gather(user_table, movie_table, uidx, midx)
  y = _mlp(uemb, memb,
           W1[:_EMBED], W1[_EMBED:], b1.reshape(1, -1),
           W2, b2.reshape(1, -1), W3, b3.reshape(1, 1))
  return y
